# Initial kernel scaffold; baseline (speedup 1.0000x reference)
import functools

import jax
import jax.numpy as jnp
from jax import lax
from jax.experimental import pallas as pl
from jax.experimental.pallas import tpu as pltpu
from jax.experimental.pallas import tpu_sc as plsc

N_NODES = 10000
N_EDGES = 320000
D = 128

NC = 2
NS = 16
NW = NC * NS
CHUNK = 128
NCHUNK = -(-N_EDGES // (NW * CHUNK))
EPAD = NW * NCHUNK * CHUNK
NPAD = ((N_NODES + NS * 8 - 1) // (NS * 8)) * (NS * 8)
ROWS_PER_TILE = NPAD // NS

_mesh = plsc.VectorSubcoreMesh(core_axis_name="c", subcore_axis_name="s")


@jax.jit
def _sc_deg(cols, ews):
  @functools.partial(
      pl.kernel,
      out_type=jax.ShapeDtypeStruct((NC, NPAD), jnp.float32),
      mesh=_mesh,
      scratch_types=[
          pltpu.VMEM((NCHUNK, CHUNK), jnp.int32),
          pltpu.VMEM((NCHUNK, CHUNK), jnp.float32),
          pltpu.VMEM((ROWS_PER_TILE,), jnp.float32),
          pltpu.VMEM_SHARED((NPAD,), jnp.float32),
      ],
  )
  def deg_kernel(cols_hbm, ews_hbm, deg_hbm, colv, ewv, zv, acc):
    cid = lax.axis_index("c")
    sid = lax.axis_index("s")
    wid = sid * NC + cid

    @pl.loop(0, ROWS_PER_TILE // 16)
    def _(i):
      zv[pl.ds(i * 16, 16)] = jnp.zeros((16,), jnp.float32)

    pltpu.sync_copy(zv, acc.at[pl.ds(sid * ROWS_PER_TILE, ROWS_PER_TILE)])
    plsc.subcore_barrier()

    pltpu.sync_copy(cols_hbm.at[wid], colv)
    pltpu.sync_copy(ews_hbm.at[wid], ewv)

    @pl.loop(0, NCHUNK)
    def _(k):
      pltpu.sync_copy(ewv.at[k], acc.at[colv.at[k]], add=True)

    plsc.subcore_barrier()
    pltpu.sync_copy(acc.at[pl.ds(sid * ROWS_PER_TILE, ROWS_PER_TILE)],
                    deg_hbm.at[cid, pl.ds(sid * ROWS_PER_TILE, ROWS_PER_TILE)])

  return deg_kernel(cols, ews)


@jax.jit
def _sc_agg(h, rows, cols, ews):
  @functools.partial(
      pl.kernel,
      out_type=jax.ShapeDtypeStruct((NC, NPAD, D), jnp.float32),
      mesh=_mesh,
      scratch_types=[
          pltpu.VMEM((CHUNK,), jnp.int32),
          pltpu.VMEM((CHUNK,), jnp.int32),
          pltpu.VMEM((CHUNK,), jnp.float32),
          pltpu.VMEM((CHUNK, D), jnp.float32),
          pltpu.VMEM((64, D), jnp.float32),
          pltpu.VMEM_SHARED((NPAD, D), jnp.float32),
      ],
  )
  def agg_kernel(h_hbm, rows_hbm, cols_hbm, ews_hbm, out_hbm,
                 rowv, colv, ewv, gbuf, zbuf, acc):
    cid = lax.axis_index("c")
    sid = lax.axis_index("s")
    wid = sid * NC + cid

    @pl.loop(0, 64)
    def _(i):
      for k in range(D // 16):
        zbuf[i, pl.ds(k * 16, 16)] = jnp.zeros((16,), jnp.float32)

    @pl.loop(0, ROWS_PER_TILE // 64)
    def _(i):
      pltpu.sync_copy(zbuf, acc.at[pl.ds(sid * ROWS_PER_TILE + i * 64, 64)])

    plsc.subcore_barrier()

    @pl.loop(0, NCHUNK)
    def _(c):
      pltpu.sync_copy(rows_hbm.at[wid, c], rowv)
      pltpu.sync_copy(cols_hbm.at[wid, c], colv)
      pltpu.sync_copy(ews_hbm.at[wid, c], ewv)
      pltpu.sync_copy(h_hbm.at[rowv], gbuf)

      @pl.loop(0, CHUNK)
      def _(j):
        idx = jnp.full((16,), j, dtype=jnp.int32)
        ew16 = plsc.load_gather(ewv, [idx])
        for k in range(D // 16):
          gbuf[j, pl.ds(k * 16, 16)] = gbuf[j, pl.ds(k * 16, 16)] * ew16

      pltpu.sync_copy(gbuf, acc.at[colv], add=True)

    plsc.subcore_barrier()

    @pl.loop(0, ROWS_PER_TILE // 64)
    def _(i):
      r = sid * ROWS_PER_TILE + i * 64
      pltpu.sync_copy(acc.at[pl.ds(r, 64)], out_hbm.at[cid, pl.ds(r, 64)])

  return agg_kernel(h, rows, cols, ews)


_BR = 1000


def _tc_matmul(x, W):
  def body(x_ref, w_ref, o_ref):
    o_ref[...] = jnp.dot(x_ref[...], w_ref[...],
                         preferred_element_type=jnp.float32)

  return pl.pallas_call(
      body,
      grid=(N_NODES // _BR,),
      in_specs=[
          pl.BlockSpec((_BR, D), lambda i: (i, 0)),
          pl.BlockSpec((D, D), lambda i: (0, 0)),
      ],
      out_specs=pl.BlockSpec((_BR, D), lambda i: (i, 0)),
      out_shape=jax.ShapeDtypeStruct((N_NODES, D), jnp.float32),
  )(x, W)


def _tc_dinv_scale(dega, degb, h):
  def body(da_ref, db_ref, h_ref, dinv_ref, ht_ref):
    deg = da_ref[...] + db_ref[...] + 1.0
    dinv = jax.lax.rsqrt(deg)
    dinv_ref[...] = dinv
    ht_ref[...] = dinv * h_ref[...]

  return pl.pallas_call(
      body,
      grid=(N_NODES // _BR,),
      in_specs=[
          pl.BlockSpec((_BR, 1), lambda i: (i, 0)),
          pl.BlockSpec((_BR, 1), lambda i: (i, 0)),
          pl.BlockSpec((_BR, D), lambda i: (i, 0)),
      ],
      out_specs=[
          pl.BlockSpec((_BR, 1), lambda i: (i, 0)),
          pl.BlockSpec((_BR, D), lambda i: (i, 0)),
      ],
      out_shape=[
          jax.ShapeDtypeStruct((N_NODES, 1), jnp.float32),
          jax.ShapeDtypeStruct((N_NODES, D), jnp.float32),
      ],
  )(dega, degb, h)


def _tc_mid(agg0, agg1, ht, dinv, b, W):
  def body(a0_ref, a1_ref, ht_ref, dinv_ref, b_ref, w_ref, o_ref):
    z = dinv_ref[...] * (a0_ref[...] + a1_ref[...] + ht_ref[...]) + b_ref[...]
    a = jnp.maximum(z, 0.0)
    o_ref[...] = dinv_ref[...] * jnp.dot(a, w_ref[...],
                                         preferred_element_type=jnp.float32)

  return pl.pallas_call(
      body,
      grid=(N_NODES // _BR,),
      in_specs=[
          pl.BlockSpec((_BR, D), lambda i: (i, 0)),
          pl.BlockSpec((_BR, D), lambda i: (i, 0)),
          pl.BlockSpec((_BR, D), lambda i: (i, 0)),
          pl.BlockSpec((_BR, 1), lambda i: (i, 0)),
          pl.BlockSpec((1, D), lambda i: (0, 0)),
          pl.BlockSpec((D, D), lambda i: (0, 0)),
      ],
      out_specs=pl.BlockSpec((_BR, D), lambda i: (i, 0)),
      out_shape=jax.ShapeDtypeStruct((N_NODES, D), jnp.float32),
  )(agg0, agg1, ht, dinv, b, W)


def _tc_final(agg0, agg1, ht, dinv, b):
  def body(a0_ref, a1_ref, ht_ref, dinv_ref, b_ref, o_ref):
    o_ref[...] = (dinv_ref[...] * (a0_ref[...] + a1_ref[...] + ht_ref[...])
                  + b_ref[...])

  return pl.pallas_call(
      body,
      grid=(N_NODES // _BR,),
      in_specs=[
          pl.BlockSpec((_BR, D), lambda i: (i, 0)),
          pl.BlockSpec((_BR, D), lambda i: (i, 0)),
          pl.BlockSpec((_BR, D), lambda i: (i, 0)),
          pl.BlockSpec((_BR, 1), lambda i: (i, 0)),
          pl.BlockSpec((1, D), lambda i: (0, 0)),
      ],
      out_specs=pl.BlockSpec((_BR, D), lambda i: (i, 0)),
      out_shape=jax.ShapeDtypeStruct((N_NODES, D), jnp.float32),
  )(agg0, agg1, ht, dinv, b)


def kernel(x, edge_index, edge_weight, W1, b1, W2, b2, W3, b3):
  pad = EPAD - N_EDGES
  rows = jnp.concatenate(
      [edge_index[0].astype(jnp.int32), jnp.zeros((pad,), jnp.int32)]
  ).reshape(NW, NCHUNK, CHUNK)
  cols = jnp.concatenate(
      [edge_index[1].astype(jnp.int32), jnp.zeros((pad,), jnp.int32)]
  ).reshape(NW, NCHUNK, CHUNK)
  ews = jnp.concatenate(
      [edge_weight, jnp.zeros((pad,), jnp.float32)]
  ).reshape(NW, NCHUNK, CHUNK)

  b1r = b1.reshape(1, D)
  b2r = b2.reshape(1, D)
  b3r = b3.reshape(1, D)

  deg = _sc_deg(cols, ews)
  h1 = _tc_matmul(x, W1)

  dega = deg[0, :N_NODES].reshape(N_NODES, 1)
  degb = deg[1, :N_NODES].reshape(N_NODES, 1)
  dinv, ht1 = _tc_dinv_scale(dega, degb, h1)

  ht1p = jnp.pad(ht1, ((0, NPAD - N_NODES), (0, 0)))
  agg1 = _sc_agg(ht1p, rows, cols, ews)
  ht2 = _tc_mid(agg1[0, :N_NODES], agg1[1, :N_NODES], ht1, dinv, b1r, W2)

  ht2p = jnp.pad(ht2, ((0, NPAD - N_NODES), (0, 0)))
  agg2 = _sc_agg(ht2p, rows, cols, ews)
  ht3 = _tc_mid(agg2[0, :N_NODES], agg2[1, :N_NODES], ht2, dinv, b2r, W3)

  ht3p = jnp.pad(ht3, ((0, NPAD - N_NODES), (0, 0)))
  agg3 = _sc_agg(ht3p, rows, cols, ews)
  return _tc_final(agg3[0, :N_NODES], agg3[1, :N_NODES], ht3, dinv, b3r)


# speedup vs baseline: 8.8018x; 8.8018x over previous
import dataclasses
import functools

import jax
import jax.numpy as jnp
from jax import lax
from jax.experimental import pallas as pl
from jax.experimental.pallas import tpu as pltpu
from jax.experimental.pallas import tpu_sc as plsc

N_NODES = 10000
N_EDGES = 320000
D = 128

NC = 2
NS = 16
NW = NC * NS
CHUNK = 128
NCHUNK = -(-N_EDGES // (NW * CHUNK))
EPAD = NW * NCHUNK * CHUNK
ECHT = NCHUNK * CHUNK
NPAD = -(-N_NODES // (NS * 128)) * (NS * 128)
ROWS_PER_TILE = NPAD // NS

_mesh = plsc.VectorSubcoreMesh(core_axis_name="c", subcore_axis_name="s")

_cp = pltpu.CompilerParams()
if "needs_layout_passes" in pltpu.CompilerParams.__dataclass_fields__:
  _cp = dataclasses.replace(_cp, needs_layout_passes=False)


@jax.jit
def _sc_deg(cols, ews):
  @functools.partial(
      pl.kernel,
      out_type=jax.ShapeDtypeStruct((NC * NPAD,), jnp.float32),
      mesh=_mesh,
      compiler_params=_cp,
      scratch_types=[
          pltpu.VMEM((NCHUNK, CHUNK), jnp.int32),
          pltpu.VMEM((ECHT,), jnp.float32),
          pltpu.VMEM((ROWS_PER_TILE,), jnp.float32),
          pltpu.VMEM_SHARED((NPAD,), jnp.float32),
      ],
  )
  def deg_kernel(cols_hbm, ews_hbm, deg_hbm, colv, ewv, zv, acc):
    cid = lax.axis_index("c")
    sid = lax.axis_index("s")
    wid = sid * NC + cid

    @pl.loop(0, ROWS_PER_TILE // 16)
    def _(i):
      zv[pl.ds(i * 16, 16)] = jnp.zeros((16,), jnp.float32)

    pltpu.sync_copy(zv, acc.at[pl.ds(sid * ROWS_PER_TILE, ROWS_PER_TILE)])
    plsc.subcore_barrier()

    pltpu.sync_copy(cols_hbm.at[wid], colv)
    pltpu.sync_copy(ews_hbm.at[wid], ewv)

    @pl.loop(0, NCHUNK)
    def _(k):
      pltpu.sync_copy(ewv.at[pl.ds(k * CHUNK, CHUNK)],
                      acc.at[colv.at[k]], add=True)

    plsc.subcore_barrier()
    pltpu.sync_copy(
        acc.at[pl.ds(sid * ROWS_PER_TILE, ROWS_PER_TILE)],
        deg_hbm.at[pl.ds(cid * NPAD + sid * ROWS_PER_TILE, ROWS_PER_TILE)])

  return deg_kernel(cols, ews)


@jax.jit
def _sc_agg(h, z, rows, cols, ews):
  @functools.partial(
      pl.kernel,
      out_type=jax.ShapeDtypeStruct((NC, NPAD, D), jnp.float32),
      mesh=_mesh,
      compiler_params=_cp,
      scratch_types=[
          pltpu.VMEM((ECHT,), jnp.int32),
          pltpu.VMEM((NCHUNK, CHUNK), jnp.int32),
          pltpu.VMEM((ECHT,), jnp.float32),
          pltpu.VMEM((CHUNK, D), jnp.float32),
          pltpu.VMEM_SHARED((NPAD, D), jnp.float32),
      ],
  )
  def agg_kernel(h_hbm, z_hbm, rows_hbm, cols_hbm, ews_hbm, out_hbm,
                 rowv, colv, ewv, gbuf, acc):
    cid = lax.axis_index("c")
    sid = lax.axis_index("s")
    wid = sid * NC + cid

    pltpu.sync_copy(z_hbm.at[pl.ds(sid * ROWS_PER_TILE, ROWS_PER_TILE)],
                    acc.at[pl.ds(sid * ROWS_PER_TILE, ROWS_PER_TILE)])

    pltpu.sync_copy(rows_hbm.at[wid], rowv)
    pltpu.sync_copy(cols_hbm.at[wid], colv)
    pltpu.sync_copy(ews_hbm.at[wid], ewv)
    plsc.subcore_barrier()

    @pl.loop(0, NCHUNK)
    def _(c):
      pltpu.sync_copy(h_hbm.at[rowv.at[pl.ds(c * CHUNK, CHUNK)]], gbuf)

      @pl.loop(0, CHUNK)
      def _(j):
        idx = jnp.full((16,), c * CHUNK + j, dtype=jnp.int32)
        ew16 = plsc.load_gather(ewv, [idx])
        for k in range(D // 16):
          gbuf[j, pl.ds(k * 16, 16)] = gbuf[j, pl.ds(k * 16, 16)] * ew16

      pltpu.sync_copy(gbuf, acc.at[colv.at[c]], add=True)

    plsc.subcore_barrier()

    @pl.loop(0, ROWS_PER_TILE // 64)
    def _(i):
      r = sid * ROWS_PER_TILE + i * 64
      pltpu.sync_copy(acc.at[pl.ds(r, 64)], out_hbm.at[cid, pl.ds(r, 64)])

  return agg_kernel(h, z, rows, cols, ews)


_BR = 1000


def _tc_matmul(x, W):
  def body(x_ref, w_ref, o_ref):
    o_ref[...] = jnp.dot(x_ref[...], w_ref[...],
                         preferred_element_type=jnp.float32)

  return pl.pallas_call(
      body,
      grid=(N_NODES // _BR,),
      in_specs=[
          pl.BlockSpec((_BR, D), lambda i: (i, 0)),
          pl.BlockSpec((D, D), lambda i: (0, 0)),
      ],
      out_specs=pl.BlockSpec((_BR, D), lambda i: (i, 0)),
      out_shape=jax.ShapeDtypeStruct((N_NODES, D), jnp.float32),
  )(x, W)


def _tc_dinv_scale(dega, degb, h):
  def body(da_ref, db_ref, h_ref, dinv_ref, ht_ref):
    deg = da_ref[...] + db_ref[...] + 1.0
    dinv = jax.lax.rsqrt(deg)
    dinv_ref[...] = dinv
    ht_ref[...] = dinv * h_ref[...]

  return pl.pallas_call(
      body,
      grid=(N_NODES // _BR,),
      in_specs=[
          pl.BlockSpec((_BR, 1), lambda i: (i, 0)),
          pl.BlockSpec((_BR, 1), lambda i: (i, 0)),
          pl.BlockSpec((_BR, D), lambda i: (i, 0)),
      ],
      out_specs=[
          pl.BlockSpec((_BR, 1), lambda i: (i, 0)),
          pl.BlockSpec((_BR, D), lambda i: (i, 0)),
      ],
      out_shape=[
          jax.ShapeDtypeStruct((N_NODES, 1), jnp.float32),
          jax.ShapeDtypeStruct((N_NODES, D), jnp.float32),
      ],
  )(dega, degb, h)


def _tc_mid(agg0, agg1, ht, dinv, b, W):
  def body(a0_ref, a1_ref, ht_ref, dinv_ref, b_ref, w_ref, o_ref):
    z = dinv_ref[...] * (a0_ref[...] + a1_ref[...] + ht_ref[...]) + b_ref[...]
    a = jnp.maximum(z, 0.0)
    o_ref[...] = dinv_ref[...] * jnp.dot(a, w_ref[...],
                                         preferred_element_type=jnp.float32)

  return pl.pallas_call(
      body,
      grid=(N_NODES // _BR,),
      in_specs=[
          pl.BlockSpec((_BR, D), lambda i: (i, 0)),
          pl.BlockSpec((_BR, D), lambda i: (i, 0)),
          pl.BlockSpec((_BR, D), lambda i: (i, 0)),
          pl.BlockSpec((_BR, 1), lambda i: (i, 0)),
          pl.BlockSpec((1, D), lambda i: (0, 0)),
          pl.BlockSpec((D, D), lambda i: (0, 0)),
      ],
      out_specs=pl.BlockSpec((_BR, D), lambda i: (i, 0)),
      out_shape=jax.ShapeDtypeStruct((N_NODES, D), jnp.float32),
  )(agg0, agg1, ht, dinv, b, W)


def _tc_final(agg0, agg1, ht, dinv, b):
  def body(a0_ref, a1_ref, ht_ref, dinv_ref, b_ref, o_ref):
    o_ref[...] = (dinv_ref[...] * (a0_ref[...] + a1_ref[...] + ht_ref[...])
                  + b_ref[...])

  return pl.pallas_call(
      body,
      grid=(N_NODES // _BR,),
      in_specs=[
          pl.BlockSpec((_BR, D), lambda i: (i, 0)),
          pl.BlockSpec((_BR, D), lambda i: (i, 0)),
          pl.BlockSpec((_BR, D), lambda i: (i, 0)),
          pl.BlockSpec((_BR, 1), lambda i: (i, 0)),
          pl.BlockSpec((1, D), lambda i: (0, 0)),
      ],
      out_specs=pl.BlockSpec((_BR, D), lambda i: (i, 0)),
      out_shape=jax.ShapeDtypeStruct((N_NODES, D), jnp.float32),
  )(agg0, agg1, ht, dinv, b)


def kernel(x, edge_index, edge_weight, W1, b1, W2, b2, W3, b3):
  pad = EPAD - N_EDGES
  rows = jnp.concatenate(
      [edge_index[0].astype(jnp.int32), jnp.zeros((pad,), jnp.int32)]
  ).reshape(NW, ECHT)
  cols = jnp.concatenate(
      [edge_index[1].astype(jnp.int32), jnp.zeros((pad,), jnp.int32)]
  ).reshape(NW, NCHUNK, CHUNK)
  ews = jnp.concatenate(
      [edge_weight, jnp.zeros((pad,), jnp.float32)]
  ).reshape(NW, ECHT)

  b1r = b1.reshape(1, D)
  b2r = b2.reshape(1, D)
  b3r = b3.reshape(1, D)

  deg = _sc_deg(cols, ews)
  h1 = _tc_matmul(x, W1)

  dega = deg[:N_NODES].reshape(N_NODES, 1)
  degb = deg[NPAD:NPAD + N_NODES].reshape(N_NODES, 1)
  dinv, ht1 = _tc_dinv_scale(dega, degb, h1)

  zeros = jnp.zeros((NPAD, D), jnp.float32)

  ht1p = jnp.pad(ht1, ((0, NPAD - N_NODES), (0, 0)))
  agg1 = _sc_agg(ht1p, zeros, rows, cols, ews)
  ht2 = _tc_mid(agg1[0, :N_NODES], agg1[1, :N_NODES], ht1, dinv, b1r, W2)

  ht2p = jnp.pad(ht2, ((0, NPAD - N_NODES), (0, 0)))
  agg2 = _sc_agg(ht2p, zeros, rows, cols, ews)
  ht3 = _tc_mid(agg2[0, :N_NODES], agg2[1, :N_NODES], ht2, dinv, b2r, W3)

  ht3p = jnp.pad(ht3, ((0, NPAD - N_NODES), (0, 0)))
  agg3 = _sc_agg(ht3p, zeros, rows, cols, ews)
  return _tc_final(agg3[0, :N_NODES], agg3[1, :N_NODES], ht3, dinv, b3r)
